# baseline (speedup 1.0000x reference)
import functools

import jax
import jax.numpy as jnp
from jax import lax
from jax.experimental import pallas as pl
from jax.experimental.pallas import tpu as pltpu
from jax.experimental.pallas import tpu_sc as plsc

NUM_VOCAB = 1000000
EMBED_DIM = 64
HIDDEN_DIM = 256
BATCH = 4096
SEQ = 200

_INFO = plsc.get_sparse_core_info()
_NC = _INFO.num_cores
_NS = _INFO.num_subcores
_NW = _NC * _NS
_BPW = BATCH // _NW
_CB = 2
_NCHUNK = _BPW // _CB
_CHUNK_IDX = _CB * SEQ
_SLICES = [(0, 128), (128, 128), (256, 128), (384, 16)]


def _sc_pool(xf, table):
  mesh = plsc.VectorSubcoreMesh(core_axis_name="c", subcore_axis_name="s")

  @functools.partial(
      pl.kernel,
      out_type=jax.ShapeDtypeStruct((BATCH, EMBED_DIM), jnp.float32),
      mesh=mesh,
      scratch_types=[
          pltpu.VMEM((_CHUNK_IDX,), jnp.int32),
          pltpu.VMEM((_CHUNK_IDX, EMBED_DIM), jnp.float32),
          pltpu.VMEM((_BPW, EMBED_DIM), jnp.float32),
          pltpu.SemaphoreType.DMA,
      ],
      compiler_params=pltpu.CompilerParams(use_tc_tiling_on_sc=False),
  )
  def k(xf_hbm, table_hbm, out_hbm, idx_v, rows_v, out_v, sem):
    wid = lax.axis_index("s") * _NC + lax.axis_index("c")
    base = wid * _BPW

    def chunk_body(c, carry):
      off = (base + c * _CB) * SEQ
      pltpu.sync_copy(xf_hbm.at[pl.ds(off, _CHUNK_IDX)], idx_v)
      descs = [
          pltpu.async_copy(
              table_hbm.at[idx_v.at[pl.ds(o, l)]],
              rows_v.at[pl.ds(o, l)],
              sem,
          )
          for o, l in _SLICES
      ]
      for d in descs:
        d.wait()
      for e in range(_CB):
        zero = jnp.zeros((16,), jnp.float32)

        @plsc.parallel_loop(0, SEQ, unroll=8, carry=(zero, zero, zero, zero))
        def accs(r, acc):
          row = e * SEQ + r
          return tuple(
              acc[g] + rows_v[row, pl.ds(g * 16, 16)] for g in range(4)
          )

        orow = c * _CB + e
        for g in range(4):
          out_v[orow, pl.ds(g * 16, 16)] = accs[g] * (1.0 / SEQ)
      return carry

    lax.fori_loop(0, _NCHUNK, chunk_body, 0)
    pltpu.sync_copy(out_v, out_hbm.at[pl.ds(base, _BPW)])

  return k(xf, table)


def _mlp_body(h0_ref, w1_ref, b1_ref, w2_ref, b2_ref, o_ref):
  h = h0_ref[...]
  h1 = lax.dot_general(
      h, w1_ref[...], (((1,), (1,)), ((), ())),
      preferred_element_type=jnp.float32,
  )
  h1 = jnp.maximum(h1 + b1_ref[...], 0.0)
  o = jnp.sum(h1 * w2_ref[...], axis=1, keepdims=True) + b2_ref[...]
  o_ref[...] = 1.0 / (1.0 + jnp.exp(-o))


def _tc_mlp(h0, W1, b1, W2, b2):
  nb = 8
  bm = BATCH // nb
  return pl.pallas_call(
      _mlp_body,
      grid=(nb,),
      in_specs=[
          pl.BlockSpec((bm, EMBED_DIM), lambda i: (i, 0)),
          pl.BlockSpec((HIDDEN_DIM, EMBED_DIM), lambda i: (0, 0)),
          pl.BlockSpec((1, HIDDEN_DIM), lambda i: (0, 0)),
          pl.BlockSpec((1, HIDDEN_DIM), lambda i: (0, 0)),
          pl.BlockSpec((1, 1), lambda i: (0, 0)),
      ],
      out_specs=pl.BlockSpec((bm, 1), lambda i: (i, 0)),
      out_shape=jax.ShapeDtypeStruct((BATCH, 1), jnp.float32),
  )(h0, W1, b1, W2, b2)


@jax.jit
def kernel(x, table, W1, b1, W2, b2):
  xf = jnp.reshape(x, (BATCH * SEQ,))
  h0 = _sc_pool(xf, table)
  out = _tc_mlp(h0, W1, b1.reshape(1, HIDDEN_DIM), W2, b2.reshape(1, 1))
  return jnp.squeeze(out, axis=1)


# speedup vs baseline: 1.1259x; 1.1259x over previous
import functools

import jax
import jax.numpy as jnp
from jax import lax
from jax.experimental import pallas as pl
from jax.experimental.pallas import tpu as pltpu
from jax.experimental.pallas import tpu_sc as plsc

NUM_VOCAB = 1000000
EMBED_DIM = 64
ROW = 128
HIDDEN_DIM = 256
BATCH = 4096
SEQ = 200

_INFO = plsc.get_sparse_core_info()
_NC = _INFO.num_cores
_NS = _INFO.num_subcores
_NW = _NC * _NS
_BPW = BATCH // _NW
_SLICES = [(0, 128), (128, 72)]


def _sc_pool(xf, tp):
  mesh = plsc.VectorSubcoreMesh(core_axis_name="c", subcore_axis_name="s")

  @functools.partial(
      pl.kernel,
      out_type=jax.ShapeDtypeStruct((BATCH, EMBED_DIM), jnp.float32),
      mesh=mesh,
      scratch_types=[
          pltpu.VMEM((SEQ,), jnp.int32),
          pltpu.VMEM((SEQ,), jnp.int32),
          pltpu.VMEM((SEQ, ROW), jnp.float32),
          pltpu.VMEM((SEQ, ROW), jnp.float32),
          pltpu.VMEM((_BPW, EMBED_DIM), jnp.float32),
          pltpu.SemaphoreType.DMA,
          pltpu.SemaphoreType.DMA,
      ],
      compiler_params=pltpu.CompilerParams(use_tc_tiling_on_sc=True),
  )
  def k(xf_hbm, tp_hbm, out_hbm, idx0, idx1, rows0, rows1, out_v, sem0, sem1):
    wid = lax.axis_index("s") * _NC + lax.axis_index("c")
    base = wid * _BPW
    bufs = ((idx0, rows0, sem0), (idx1, rows1, sem1))

    def start(c, idx_v, rows_v, sem):
      off = (base + c) * SEQ
      pltpu.sync_copy(xf_hbm.at[pl.ds(off, SEQ)], idx_v)
      for o, l in _SLICES:
        pltpu.async_copy(
            tp_hbm.at[idx_v.at[pl.ds(o, l)]], rows_v.at[pl.ds(o, l)], sem
        )

    def finish(c, idx_v, rows_v, sem):
      for o, l in _SLICES:
        pltpu.make_async_copy(
            tp_hbm.at[idx_v.at[pl.ds(o, l)]], rows_v.at[pl.ds(o, l)], sem
        ).wait()
      zero = jnp.zeros((16,), jnp.float32)

      @plsc.parallel_loop(0, SEQ, unroll=8, carry=(zero, zero, zero, zero))
      def accs(r, acc):
        return tuple(
            acc[g] + rows_v[r, pl.ds(g * 16, 16)] for g in range(4)
        )

      for g in range(4):
        out_v[c, pl.ds(g * 16, 16)] = accs[g] * (1.0 / SEQ)

    for b in range(2):
      start(b, *bufs[b])

    def chunk_body(g, carry):
      for b in range(2):
        c = 2 * g + b
        idx_v, rows_v, sem = bufs[b]
        finish(c, idx_v, rows_v, sem)

        @pl.when(c + 2 < _BPW)
        def _():
          start(c + 2, idx_v, rows_v, sem)

      return carry

    lax.fori_loop(0, _BPW // 2, chunk_body, 0)
    pltpu.sync_copy(out_v, out_hbm.at[pl.ds(base, _BPW)])

  return k(xf, tp)


def _mlp_body(h0_ref, w1_ref, b1_ref, w2_ref, b2_ref, o_ref):
  h = h0_ref[...]
  h1 = lax.dot_general(
      h, w1_ref[...], (((1,), (1,)), ((), ())),
      preferred_element_type=jnp.float32,
  )
  h1 = jnp.maximum(h1 + b1_ref[...], 0.0)
  o = jnp.sum(h1 * w2_ref[...], axis=1, keepdims=True) + b2_ref[...]
  o_ref[...] = 1.0 / (1.0 + jnp.exp(-o))


def _tc_mlp(h0, W1, b1, W2, b2):
  nb = 8
  bm = BATCH // nb
  return pl.pallas_call(
      _mlp_body,
      grid=(nb,),
      in_specs=[
          pl.BlockSpec((bm, EMBED_DIM), lambda i: (i, 0)),
          pl.BlockSpec((HIDDEN_DIM, EMBED_DIM), lambda i: (0, 0)),
          pl.BlockSpec((1, HIDDEN_DIM), lambda i: (0, 0)),
          pl.BlockSpec((1, HIDDEN_DIM), lambda i: (0, 0)),
          pl.BlockSpec((1, 1), lambda i: (0, 0)),
      ],
      out_specs=pl.BlockSpec((bm, 1), lambda i: (i, 0)),
      out_shape=jax.ShapeDtypeStruct((BATCH, 1), jnp.float32),
  )(h0, W1, b1, W2, b2)


@jax.jit
def kernel(x, table, W1, b1, W2, b2):
  xf = jnp.reshape(x, (BATCH * SEQ,))
  tp = jnp.pad(table, ((0, 0), (0, ROW - EMBED_DIM)))
  h0 = _sc_pool(xf, tp)
  out = _tc_mlp(h0, W1, b1.reshape(1, HIDDEN_DIM), W2, b2.reshape(1, 1))
  return jnp.squeeze(out, axis=1)
